# Initial kernel scaffold; baseline (speedup 1.0000x reference)
import functools

import jax
import jax.numpy as jnp
from jax import lax
from jax.experimental import pallas as pl
from jax.experimental.pallas import tpu as pltpu
from jax.experimental.pallas import tpu_sc as plsc

N = 10000
E = 320000
D = 128

NC = 2
NS = 16
NW = NC * NS
NPAD = 10240
RPS = NPAD // NS
EPW = E // NW
CHUNK = 128
NFULL = EPW // CHUNK
TAIL = EPW - NFULL * CHUNK
DEGW = 16


def _sc_mesh():
    return plsc.VectorSubcoreMesh(core_axis_name="c", subcore_axis_name="s")


def _deg_sc(edge_index):
    @functools.partial(
        pl.kernel,
        out_type=jax.ShapeDtypeStruct((NC, NPAD, DEGW), jnp.float32),
        mesh=_sc_mesh(),
        scratch_types=[
            pltpu.VMEM((CHUNK,), jnp.int32),
            pltpu.VMEM((TAIL,), jnp.int32),
            pltpu.VMEM((CHUNK, DEGW), jnp.float32),
            pltpu.VMEM((TAIL, DEGW), jnp.float32),
            pltpu.VMEM((RPS, DEGW), jnp.float32),
            pltpu.VMEM_SHARED((NPAD, DEGW), jnp.float32),
        ],
    )
    def k(ei_hbm, out_hbm, di, di_t, ones_v, ones_t, zbuf, acc):
        c = lax.axis_index("c")
        s = lax.axis_index("s")
        base = (c * NS + s) * EPW

        def _fill(r, carry):
            zbuf[r, :] = jnp.zeros((16,), jnp.float32)
            return carry

        lax.fori_loop(0, RPS, _fill, 0)

        def _fill1(r, carry):
            ones_v[r, :] = jnp.ones((16,), jnp.float32)
            return carry

        lax.fori_loop(0, CHUNK, _fill1, 0)
        for r in range(TAIL):
            ones_t[r, :] = jnp.ones((16,), jnp.float32)

        pltpu.sync_copy(zbuf, acc.at[pl.ds(s * RPS, RPS), :])
        plsc.subcore_barrier()

        def _body(i, carry):
            off = base + i * CHUNK
            pltpu.sync_copy(ei_hbm.at[1, pl.ds(off, CHUNK)], di)
            pltpu.sync_copy(ones_v, acc.at[di], add=True)
            return carry

        lax.fori_loop(0, NFULL, _body, 0)
        offt = base + NFULL * CHUNK
        pltpu.sync_copy(ei_hbm.at[1, pl.ds(offt, TAIL)], di_t)
        pltpu.sync_copy(ones_t, acc.at[di_t], add=True)

        plsc.subcore_barrier()
        pltpu.sync_copy(acc.at[pl.ds(s * RPS, RPS), :],
                        out_hbm.at[c, pl.ds(s * RPS, RPS), :])

    return k(edge_index)


def _seg_sc(edge_index, g):
    @functools.partial(
        pl.kernel,
        out_type=jax.ShapeDtypeStruct((NC, NPAD, D), jnp.float32),
        mesh=_sc_mesh(),
        scratch_types=[
            pltpu.VMEM((CHUNK,), jnp.int32),
            pltpu.VMEM((CHUNK,), jnp.int32),
            pltpu.VMEM((TAIL,), jnp.int32),
            pltpu.VMEM((TAIL,), jnp.int32),
            pltpu.VMEM((CHUNK, D), jnp.float32),
            pltpu.VMEM((TAIL, D), jnp.float32),
            pltpu.VMEM((CHUNK, D), jnp.float32),
            pltpu.VMEM_SHARED((NPAD, D), jnp.float32),
        ],
    )
    def k(ei_hbm, g_hbm, out_hbm, si, di, si_t, di_t, rows, rows_t, zbuf, acc):
        c = lax.axis_index("c")
        s = lax.axis_index("s")
        base = (c * NS + s) * EPW

        def _fill(r, carry):
            for q in range(D // 16):
                zbuf[r, pl.ds(q * 16, 16)] = jnp.zeros((16,), jnp.float32)
            return carry

        lax.fori_loop(0, CHUNK, _fill, 0)
        for j in range(RPS // CHUNK):
            pltpu.sync_copy(zbuf, acc.at[pl.ds(s * RPS + j * CHUNK, CHUNK), :])
        plsc.subcore_barrier()

        def _body(i, carry):
            off = base + i * CHUNK
            pltpu.sync_copy(ei_hbm.at[0, pl.ds(off, CHUNK)], si)
            pltpu.sync_copy(ei_hbm.at[1, pl.ds(off, CHUNK)], di)
            pltpu.sync_copy(g_hbm.at[si], rows)
            pltpu.sync_copy(rows, acc.at[di], add=True)
            return carry

        lax.fori_loop(0, NFULL, _body, 0)
        offt = base + NFULL * CHUNK
        pltpu.sync_copy(ei_hbm.at[0, pl.ds(offt, TAIL)], si_t)
        pltpu.sync_copy(ei_hbm.at[1, pl.ds(offt, TAIL)], di_t)
        pltpu.sync_copy(g_hbm.at[si_t], rows_t)
        pltpu.sync_copy(rows_t, acc.at[di_t], add=True)

        plsc.subcore_barrier()
        pltpu.sync_copy(acc.at[pl.ds(s * RPS, RPS), :],
                        out_hbm.at[c, pl.ds(s * RPS, RPS), :])

    return k(edge_index, g)


def _tc1_body(degp_ref, x_ref, w1_ref, dinv_ref, g1_ref):
    deg = degp_ref[0, :N, 0:1] + degp_ref[1, :N, 0:1] + 1.0
    dinv = lax.rsqrt(deg)
    dinv_ref[...] = dinv
    h = jnp.dot(x_ref[...], w1_ref[...], preferred_element_type=jnp.float32)
    g1_ref[...] = dinv * h


def _tc_first_body(g_ref, segp_ref, dinv_ref, b_ref, w_ref, h_ref, gn_ref):
    seg = segp_ref[0, :N, :] + segp_ref[1, :N, :]
    dinv = dinv_ref[...]
    conv = dinv * (g_ref[...] + seg) + b_ref[...]
    h = jnp.maximum(conv, 0.0)
    h_ref[...] = h
    gn_ref[...] = dinv * jnp.dot(h, w_ref[...], preferred_element_type=jnp.float32)


def _tc_mid_body(hp_ref, g_ref, segp_ref, dinv_ref, b_ref, w_ref, h_ref, gn_ref):
    seg = segp_ref[0, :N, :] + segp_ref[1, :N, :]
    dinv = dinv_ref[...]
    conv = dinv * (g_ref[...] + seg) + b_ref[...]
    h = jnp.maximum(hp_ref[...] + conv, 0.0)
    h_ref[...] = h
    gn_ref[...] = dinv * jnp.dot(h, w_ref[...], preferred_element_type=jnp.float32)


def _tc_out_body(hp_ref, g_ref, segp_ref, dinv_ref, b_ref,
                 wh1_ref, bh1_ref, wh2_ref, bh2_ref, out_ref):
    seg = segp_ref[0, :N, :] + segp_ref[1, :N, :]
    dinv = dinv_ref[...]
    conv = dinv * (g_ref[...] + seg) + b_ref[...]
    h = jnp.maximum(hp_ref[...] + conv, 0.0)
    z = jnp.dot(h, wh1_ref[...], preferred_element_type=jnp.float32) + bh1_ref[...]
    out_ref[...] = jnp.dot(z, wh2_ref[...], preferred_element_type=jnp.float32) + bh2_ref[...]


def _sds(shape):
    return jax.ShapeDtypeStruct(shape, jnp.float32)


def kernel(x, edge_index, W1, b1, W2, b2, W3, b3, Wh1, bh1, Wh2, bh2):
    degp = _deg_sc(edge_index)
    dinv, g1 = pl.pallas_call(
        _tc1_body, out_shape=[_sds((N, 1)), _sds((N, D))],
    )(degp, x, W1)
    segp1 = _seg_sc(edge_index, g1)
    h1, g2 = pl.pallas_call(
        _tc_first_body, out_shape=[_sds((N, D)), _sds((N, D))],
    )(g1, segp1, dinv, b1, W2)
    segp2 = _seg_sc(edge_index, g2)
    h2, g3 = pl.pallas_call(
        _tc_mid_body, out_shape=[_sds((N, D)), _sds((N, D))],
    )(h1, g2, segp2, dinv, b2, W3)
    segp3 = _seg_sc(edge_index, g3)
    out = pl.pallas_call(
        _tc_out_body, out_shape=_sds((N, 1)),
    )(h2, g3, segp3, dinv, b3, Wh1, bh1, Wh2, bh2)
    return out


# speedup vs baseline: 13.4414x; 13.4414x over previous
import functools

import jax
import jax.numpy as jnp
from jax import lax
from jax.experimental import pallas as pl
from jax.experimental.pallas import tpu as pltpu
from jax.experimental.pallas import tpu_sc as plsc

N = 10000
E = 320000
D = 128

NC = 2
NS = 16
NW = NC * NS
NPAD = 10240
RPS = NPAD // NS
EPW = E // NW
CHUNK = 128
NFULL = EPW // CHUNK
TAIL = EPW - NFULL * CHUNK
DEGW = 128


def _sc_mesh():
    return plsc.VectorSubcoreMesh(core_axis_name="c", subcore_axis_name="s")


def _deg_sc(dst):
    @functools.partial(
        pl.kernel,
        out_type=jax.ShapeDtypeStruct((NC, NPAD, DEGW), jnp.float32),
        mesh=_sc_mesh(),
        scratch_types=[
            pltpu.VMEM((CHUNK,), jnp.int32),
            pltpu.VMEM((TAIL,), jnp.int32),
            pltpu.VMEM((CHUNK, DEGW), jnp.float32),
            pltpu.VMEM((TAIL, DEGW), jnp.float32),
            pltpu.VMEM((CHUNK, DEGW), jnp.float32),
            pltpu.VMEM_SHARED((NPAD, DEGW), jnp.float32),
        ],
    )
    def k(dst_hbm, out_hbm, di, di_t, ones_v, ones_t, zbuf, acc):
        c = lax.axis_index("c")
        s = lax.axis_index("s")
        base = (c * NS + s) * EPW

        e0 = jnp.where(lax.iota(jnp.int32, 16) == 0,
                       jnp.float32(1.0), jnp.float32(0.0))

        def _fill(r, carry):
            for q in range(DEGW // 16):
                zbuf[r, pl.ds(q * 16, 16)] = jnp.zeros((16,), jnp.float32)
                ones_v[r, pl.ds(q * 16, 16)] = e0 if q == 0 else jnp.zeros((16,), jnp.float32)
            return carry

        lax.fori_loop(0, CHUNK, _fill, 0)
        for r in range(TAIL):
            for q in range(DEGW // 16):
                ones_t[r, pl.ds(q * 16, 16)] = e0 if q == 0 else jnp.zeros((16,), jnp.float32)

        for j in range(RPS // CHUNK):
            pltpu.sync_copy(zbuf, acc.at[pl.ds(s * RPS + j * CHUNK, CHUNK), :])
        plsc.subcore_barrier()

        def _body(i, carry):
            off = base + i * CHUNK
            pltpu.sync_copy(dst_hbm.at[pl.ds(off, CHUNK)], di)
            pltpu.sync_copy(ones_v, acc.at[di], add=True)
            return carry

        lax.fori_loop(0, NFULL, _body, 0)
        offt = base + NFULL * CHUNK
        pltpu.sync_copy(dst_hbm.at[pl.ds(offt, TAIL)], di_t)
        pltpu.sync_copy(ones_t, acc.at[di_t], add=True)

        plsc.subcore_barrier()
        for j in range(RPS // CHUNK):
            pltpu.sync_copy(acc.at[pl.ds(s * RPS + j * CHUNK, CHUNK), :], zbuf)
            pltpu.sync_copy(zbuf, out_hbm.at[c, pl.ds(s * RPS + j * CHUNK, CHUNK), :])

    return k(dst)


def _seg_sc(src, dst, g):
    @functools.partial(
        pl.kernel,
        out_type=jax.ShapeDtypeStruct((NC, NPAD, D), jnp.float32),
        mesh=_sc_mesh(),
        scratch_types=[
            pltpu.VMEM((CHUNK,), jnp.int32),
            pltpu.VMEM((CHUNK,), jnp.int32),
            pltpu.VMEM((TAIL,), jnp.int32),
            pltpu.VMEM((TAIL,), jnp.int32),
            pltpu.VMEM((CHUNK, D), jnp.float32),
            pltpu.VMEM((TAIL, D), jnp.float32),
            pltpu.VMEM((CHUNK, D), jnp.float32),
            pltpu.VMEM_SHARED((NPAD, D), jnp.float32),
            pltpu.SemaphoreType.DMA,
        ],
    )
    def k(src_hbm, dst_hbm, g_hbm, out_hbm, si, di, si_t, di_t, rows, rows_t,
          zbuf, acc, sem):
        c = lax.axis_index("c")
        s = lax.axis_index("s")
        base = (c * NS + s) * EPW

        def _fill(r, carry):
            for q in range(D // 16):
                zbuf[r, pl.ds(q * 16, 16)] = jnp.zeros((16,), jnp.float32)
            return carry

        lax.fori_loop(0, CHUNK, _fill, 0)
        for j in range(RPS // CHUNK):
            pltpu.sync_copy(zbuf, acc.at[pl.ds(s * RPS + j * CHUNK, CHUNK), :])
        plsc.subcore_barrier()

        def _body(i, carry):
            off = base + i * CHUNK
            pltpu.sync_copy(src_hbm.at[pl.ds(off, CHUNK)], si)
            pltpu.sync_copy(dst_hbm.at[pl.ds(off, CHUNK)], di)
            pltpu.async_copy(g_hbm.at[si], rows, sem).wait()
            pltpu.sync_copy(rows, acc.at[di], add=True)
            return carry

        lax.fori_loop(0, NFULL, _body, 0)
        offt = base + NFULL * CHUNK
        pltpu.sync_copy(src_hbm.at[pl.ds(offt, TAIL)], si_t)
        pltpu.sync_copy(dst_hbm.at[pl.ds(offt, TAIL)], di_t)
        pltpu.async_copy(g_hbm.at[si_t], rows_t, sem).wait()
        pltpu.sync_copy(rows_t, acc.at[di_t], add=True)

        plsc.subcore_barrier()
        for j in range(RPS // CHUNK):
            pltpu.sync_copy(acc.at[pl.ds(s * RPS + j * CHUNK, CHUNK), :], zbuf)
            pltpu.sync_copy(zbuf, out_hbm.at[c, pl.ds(s * RPS + j * CHUNK, CHUNK), :])

    return k(src, dst, g)


def _tc1_body(degp_ref, x_ref, w1_ref, dinv_ref, g1_ref):
    deg = degp_ref[0, :N, 0:1] + degp_ref[1, :N, 0:1] + 1.0
    dinv = lax.rsqrt(deg)
    dinv_ref[...] = dinv
    h = jnp.dot(x_ref[...], w1_ref[...], preferred_element_type=jnp.float32)
    g1_ref[...] = dinv * h


def _tc_first_body(g_ref, segp_ref, dinv_ref, b_ref, w_ref, h_ref, gn_ref):
    seg = segp_ref[0, :N, :] + segp_ref[1, :N, :]
    dinv = dinv_ref[...]
    conv = dinv * (g_ref[...] + seg) + b_ref[...]
    h = jnp.maximum(conv, 0.0)
    h_ref[...] = h
    gn_ref[...] = dinv * jnp.dot(h, w_ref[...], preferred_element_type=jnp.float32)


def _tc_mid_body(hp_ref, g_ref, segp_ref, dinv_ref, b_ref, w_ref, h_ref, gn_ref):
    seg = segp_ref[0, :N, :] + segp_ref[1, :N, :]
    dinv = dinv_ref[...]
    conv = dinv * (g_ref[...] + seg) + b_ref[...]
    h = jnp.maximum(hp_ref[...] + conv, 0.0)
    h_ref[...] = h
    gn_ref[...] = dinv * jnp.dot(h, w_ref[...], preferred_element_type=jnp.float32)


def _tc_out_body(hp_ref, g_ref, segp_ref, dinv_ref, b_ref,
                 wh1_ref, bh1_ref, wh2_ref, bh2_ref, out_ref):
    seg = segp_ref[0, :N, :] + segp_ref[1, :N, :]
    dinv = dinv_ref[...]
    conv = dinv * (g_ref[...] + seg) + b_ref[...]
    h = jnp.maximum(hp_ref[...] + conv, 0.0)
    z = jnp.dot(h, wh1_ref[...], preferred_element_type=jnp.float32) + bh1_ref[...]
    out_ref[...] = jnp.dot(z, wh2_ref[...], preferred_element_type=jnp.float32) + bh2_ref[...]


def _sds(shape):
    return jax.ShapeDtypeStruct(shape, jnp.float32)


def kernel(x, edge_index, W1, b1, W2, b2, W3, b3, Wh1, bh1, Wh2, bh2):
    src = edge_index[0]
    dst = edge_index[1]
    degp = _deg_sc(dst)
    dinv, g1 = pl.pallas_call(
        _tc1_body, out_shape=[_sds((N, 1)), _sds((N, D))],
    )(degp, x, W1)
    segp1 = _seg_sc(src, dst, g1)
    h1, g2 = pl.pallas_call(
        _tc_first_body, out_shape=[_sds((N, D)), _sds((N, D))],
    )(g1, segp1, dinv, b1, W2)
    segp2 = _seg_sc(src, dst, g2)
    h2, g3 = pl.pallas_call(
        _tc_mid_body, out_shape=[_sds((N, D)), _sds((N, D))],
    )(h1, g2, segp2, dinv, b2, W3)
    segp3 = _seg_sc(src, dst, g3)
    out = pl.pallas_call(
        _tc_out_body, out_shape=_sds((N, 1)),
    )(h2, g3, segp3, dinv, b3, Wh1, bh1, Wh2, bh2)
    return out


# speedup vs baseline: 21.6263x; 1.6089x over previous
import functools

import jax
import jax.numpy as jnp
from jax import lax
from jax.experimental import pallas as pl
from jax.experimental.pallas import tpu as pltpu
from jax.experimental.pallas import tpu_sc as plsc

N = 10000
E = 320000
D = 128

NC = 2
NS = 16
NW = NC * NS
NPAD = 10240
RPS = NPAD // NS
EPW = E // NW
CHUNK = 128
NFULL = EPW // CHUNK
TAIL = EPW - NFULL * CHUNK
DEGW = 128


def _sc_mesh():
    return plsc.VectorSubcoreMesh(core_axis_name="c", subcore_axis_name="s")


def _deg_sc(dst):
    @functools.partial(
        pl.kernel,
        out_type=jax.ShapeDtypeStruct((NC, NPAD, DEGW), jnp.float32),
        mesh=_sc_mesh(),
        scratch_types=[
            pltpu.VMEM((CHUNK,), jnp.int32),
            pltpu.VMEM((TAIL,), jnp.int32),
            pltpu.VMEM((CHUNK, DEGW), jnp.float32),
            pltpu.VMEM((TAIL, DEGW), jnp.float32),
            pltpu.VMEM((CHUNK, DEGW), jnp.float32),
            pltpu.VMEM_SHARED((NPAD, DEGW), jnp.float32),
        ],
    )
    def k(dst_hbm, out_hbm, di, di_t, ones_v, ones_t, zbuf, acc):
        c = lax.axis_index("c")
        s = lax.axis_index("s")
        base = (c * NS + s) * EPW

        e0 = jnp.where(lax.iota(jnp.int32, 16) == 0,
                       jnp.float32(1.0), jnp.float32(0.0))

        def _fill(r, carry):
            for q in range(DEGW // 16):
                zbuf[r, pl.ds(q * 16, 16)] = jnp.zeros((16,), jnp.float32)
                ones_v[r, pl.ds(q * 16, 16)] = e0 if q == 0 else jnp.zeros((16,), jnp.float32)
            return carry

        lax.fori_loop(0, CHUNK, _fill, 0)
        for r in range(TAIL):
            for q in range(DEGW // 16):
                ones_t[r, pl.ds(q * 16, 16)] = e0 if q == 0 else jnp.zeros((16,), jnp.float32)

        for j in range(RPS // CHUNK):
            pltpu.sync_copy(zbuf, acc.at[pl.ds(s * RPS + j * CHUNK, CHUNK), :])
        plsc.subcore_barrier()

        def _body(i, carry):
            off = base + i * CHUNK
            pltpu.sync_copy(dst_hbm.at[pl.ds(off, CHUNK)], di)
            pltpu.sync_copy(ones_v, acc.at[di], add=True)
            return carry

        lax.fori_loop(0, NFULL, _body, 0)
        offt = base + NFULL * CHUNK
        pltpu.sync_copy(dst_hbm.at[pl.ds(offt, TAIL)], di_t)
        pltpu.sync_copy(ones_t, acc.at[di_t], add=True)

        plsc.subcore_barrier()
        for j in range(RPS // CHUNK):
            pltpu.sync_copy(acc.at[pl.ds(s * RPS + j * CHUNK, CHUNK), :], zbuf)
            pltpu.sync_copy(zbuf, out_hbm.at[c, pl.ds(s * RPS + j * CHUNK, CHUNK), :])

    return k(dst)


def _seg_sc(src, dst, g):
    @functools.partial(
        pl.kernel,
        out_type=jax.ShapeDtypeStruct((NC, NPAD, D), jnp.float32),
        mesh=_sc_mesh(),
        scratch_types=[
            pltpu.VMEM((CHUNK,), jnp.int32),
            pltpu.VMEM((CHUNK,), jnp.int32),
            pltpu.VMEM((CHUNK,), jnp.int32),
            pltpu.VMEM((CHUNK,), jnp.int32),
            pltpu.VMEM((CHUNK, D), jnp.float32),
            pltpu.VMEM((CHUNK, D), jnp.float32),
            pltpu.VMEM((TAIL,), jnp.int32),
            pltpu.VMEM((TAIL,), jnp.int32),
            pltpu.VMEM((TAIL, D), jnp.float32),
            pltpu.VMEM_SHARED((NPAD, D), jnp.float32),
            pltpu.SemaphoreType.DMA,
            pltpu.SemaphoreType.DMA,
            pltpu.SemaphoreType.DMA,
            pltpu.SemaphoreType.DMA,
            pltpu.SemaphoreType.DMA,
            pltpu.SemaphoreType.DMA,
        ],
    )
    def k(src_hbm, dst_hbm, g_hbm, out_hbm, si0, si1, di0, di1, rows0, rows1,
          si_t, di_t, rows_t, acc, sg0, sg1, ss0, ss1, sd0, sd1):
        zbuf = rows0
        c = lax.axis_index("c")
        s = lax.axis_index("s")
        base = (c * NS + s) * EPW
        sis, dis, rowss = (si0, si1), (di0, di1), (rows0, rows1)
        sgs, sss, sds = (sg0, sg1), (ss0, ss1), (sd0, sd1)

        def _fill(r, carry):
            for q in range(D // 16):
                zbuf[r, pl.ds(q * 16, 16)] = jnp.zeros((16,), jnp.float32)
            return carry

        lax.fori_loop(0, CHUNK, _fill, 0)
        for j in range(RPS // CHUNK):
            pltpu.sync_copy(zbuf, acc.at[pl.ds(s * RPS + j * CHUNK, CHUNK), :])
        plsc.subcore_barrier()

        def _idx_start(i, b):
            off = base + i * CHUNK
            pltpu.async_copy(src_hbm.at[pl.ds(off, CHUNK)], sis[b], sss[b])
            pltpu.async_copy(dst_hbm.at[pl.ds(off, CHUNK)], dis[b], sds[b])

        def _idx_wait(i, b):
            off = base + i * CHUNK
            pltpu.make_async_copy(src_hbm.at[pl.ds(off, CHUNK)], sis[b], sss[b]).wait()
            pltpu.make_async_copy(dst_hbm.at[pl.ds(off, CHUNK)], dis[b], sds[b]).wait()

        _idx_start(0, 0)
        _idx_start(1, 1)
        _idx_wait(0, 0)
        pltpu.async_copy(g_hbm.at[sis[0]], rowss[0], sgs[0])

        def _step(i, b):
            nb = 1 - b
            pltpu.make_async_copy(g_hbm.at[sis[b]], rowss[b], sgs[b]).wait()

            @pl.when(i + 1 < NFULL)
            def _():
                _idx_wait(i + 1, nb)
                pltpu.async_copy(g_hbm.at[sis[nb]], rowss[nb], sgs[nb])

            pltpu.sync_copy(rowss[b], acc.at[dis[b]], add=True)

            @pl.when(i + 2 < NFULL)
            def _():
                _idx_start(i + 2, b)

        def _body(i2, carry):
            _step(2 * i2, 0)
            _step(2 * i2 + 1, 1)
            return carry

        lax.fori_loop(0, NFULL // 2, _body, 0)

        offt = base + NFULL * CHUNK
        pltpu.sync_copy(src_hbm.at[pl.ds(offt, TAIL)], si_t)
        pltpu.sync_copy(dst_hbm.at[pl.ds(offt, TAIL)], di_t)
        pltpu.async_copy(g_hbm.at[si_t], rows_t, sgs[0]).wait()
        pltpu.sync_copy(rows_t, acc.at[di_t], add=True)

        plsc.subcore_barrier()
        for j in range(RPS // CHUNK):
            pltpu.sync_copy(acc.at[pl.ds(s * RPS + j * CHUNK, CHUNK), :], zbuf)
            pltpu.sync_copy(zbuf, out_hbm.at[c, pl.ds(s * RPS + j * CHUNK, CHUNK), :])

    return k(src, dst, g)


def _tc1_body(degp_ref, x_ref, w1_ref, dinv_ref, g1_ref):
    deg = degp_ref[0, :N, 0:1] + degp_ref[1, :N, 0:1] + 1.0
    dinv = lax.rsqrt(deg)
    dinv_ref[...] = dinv
    h = jnp.dot(x_ref[...], w1_ref[...], preferred_element_type=jnp.float32)
    g1_ref[...] = dinv * h


def _tc_first_body(g_ref, segp_ref, dinv_ref, b_ref, w_ref, h_ref, gn_ref):
    seg = segp_ref[0, :N, :] + segp_ref[1, :N, :]
    dinv = dinv_ref[...]
    conv = dinv * (g_ref[...] + seg) + b_ref[...]
    h = jnp.maximum(conv, 0.0)
    h_ref[...] = h
    gn_ref[...] = dinv * jnp.dot(h, w_ref[...], preferred_element_type=jnp.float32)


def _tc_mid_body(hp_ref, g_ref, segp_ref, dinv_ref, b_ref, w_ref, h_ref, gn_ref):
    seg = segp_ref[0, :N, :] + segp_ref[1, :N, :]
    dinv = dinv_ref[...]
    conv = dinv * (g_ref[...] + seg) + b_ref[...]
    h = jnp.maximum(hp_ref[...] + conv, 0.0)
    h_ref[...] = h
    gn_ref[...] = dinv * jnp.dot(h, w_ref[...], preferred_element_type=jnp.float32)


def _tc_out_body(hp_ref, g_ref, segp_ref, dinv_ref, b_ref,
                 wh1_ref, bh1_ref, wh2_ref, bh2_ref, out_ref):
    seg = segp_ref[0, :N, :] + segp_ref[1, :N, :]
    dinv = dinv_ref[...]
    conv = dinv * (g_ref[...] + seg) + b_ref[...]
    h = jnp.maximum(hp_ref[...] + conv, 0.0)
    z = jnp.dot(h, wh1_ref[...], preferred_element_type=jnp.float32) + bh1_ref[...]
    out_ref[...] = jnp.dot(z, wh2_ref[...], preferred_element_type=jnp.float32) + bh2_ref[...]


def _sds(shape):
    return jax.ShapeDtypeStruct(shape, jnp.float32)


def kernel(x, edge_index, W1, b1, W2, b2, W3, b3, Wh1, bh1, Wh2, bh2):
    src = edge_index[0]
    dst = edge_index[1]
    degp = _deg_sc(dst)
    dinv, g1 = pl.pallas_call(
        _tc1_body, out_shape=[_sds((N, 1)), _sds((N, D))],
    )(degp, x, W1)
    segp1 = _seg_sc(src, dst, g1)
    h1, g2 = pl.pallas_call(
        _tc_first_body, out_shape=[_sds((N, D)), _sds((N, D))],
    )(g1, segp1, dinv, b1, W2)
    segp2 = _seg_sc(src, dst, g2)
    h2, g3 = pl.pallas_call(
        _tc_mid_body, out_shape=[_sds((N, D)), _sds((N, D))],
    )(h1, g2, segp2, dinv, b2, W3)
    segp3 = _seg_sc(src, dst, g3)
    out = pl.pallas_call(
        _tc_out_body, out_shape=_sds((N, 1)),
    )(h2, g3, segp3, dinv, b3, Wh1, bh1, Wh2, bh2)
    return out


# speedup vs baseline: 22.9952x; 1.0633x over previous
import functools

import jax
import jax.numpy as jnp
from jax import lax
from jax.experimental import pallas as pl
from jax.experimental.pallas import tpu as pltpu
from jax.experimental.pallas import tpu_sc as plsc

N = 10000
E = 320000
D = 128

NC = 2
NS = 16
NW = NC * NS
NPAD = 10240
RPS = NPAD // NS
EPW = E // NW
CHUNK = 128
NFULL = EPW // CHUNK
TAIL = EPW - NFULL * CHUNK
MAIN = (NFULL // 4) * 4
DEGW = 128


def _sc_mesh():
    return plsc.VectorSubcoreMesh(core_axis_name="c", subcore_axis_name="s")


def _deg_sc(dst):
    @functools.partial(
        pl.kernel,
        out_type=jax.ShapeDtypeStruct((NC, NPAD, DEGW), jnp.float32),
        mesh=_sc_mesh(),
        scratch_types=[
            pltpu.VMEM((CHUNK,), jnp.int32),
            pltpu.VMEM((CHUNK,), jnp.int32),
            pltpu.VMEM((CHUNK,), jnp.int32),
            pltpu.VMEM((CHUNK,), jnp.int32),
            pltpu.VMEM((TAIL,), jnp.int32),
            pltpu.VMEM((CHUNK, DEGW), jnp.float32),
            pltpu.VMEM((TAIL, DEGW), jnp.float32),
            pltpu.VMEM_SHARED((NPAD, DEGW), jnp.float32),
            pltpu.SemaphoreType.DMA,
            pltpu.SemaphoreType.DMA,
            pltpu.SemaphoreType.DMA,
            pltpu.SemaphoreType.DMA,
            pltpu.SemaphoreType.DMA,
            pltpu.SemaphoreType.DMA,
        ],
    )
    def k(dst_hbm, out_hbm, di0, di1, di2, di3, di_t, ones_v, ones_t, acc,
          sc0, sc1, sd0, sd1, sd2, sd3):
        c = lax.axis_index("c")
        s = lax.axis_index("s")
        base = (c * NS + s) * EPW
        dis, sds, scs = (di0, di1, di2, di3), (sd0, sd1, sd2, sd3), (sc0, sc1)

        e0 = jnp.where(lax.iota(jnp.int32, 16) == 0,
                       jnp.float32(1.0), jnp.float32(0.0))

        def _fill(r, carry):
            for q in range(DEGW // 16):
                ones_v[r, pl.ds(q * 16, 16)] = e0 if q == 0 else jnp.zeros(
                    (16,), jnp.float32)
            return carry

        lax.fori_loop(0, CHUNK, _fill, 0)
        for r in range(TAIL):
            for q in range(DEGW // 16):
                ones_t[r, pl.ds(q * 16, 16)] = e0 if q == 0 else jnp.zeros(
                    (16,), jnp.float32)

        def _zrow(r, carry):
            for q in range(DEGW // 16):
                ones_t[r, pl.ds(q * 16, 16)] = jnp.zeros((16,), jnp.float32)
            return carry

        lax.fori_loop(0, TAIL, _zrow, 0)
        for j in range(RPS // TAIL):
            pltpu.sync_copy(ones_t, acc.at[pl.ds(s * RPS + j * TAIL, TAIL), :])

        for r in range(TAIL):
            ones_t[r, pl.ds(0, 16)] = e0
        plsc.subcore_barrier()

        def _idx_start(i, b4):
            pltpu.async_copy(dst_hbm.at[pl.ds(base + i * CHUNK, CHUNK)],
                             dis[b4], sds[b4])

        def _idx_wait(i, b4):
            pltpu.make_async_copy(dst_hbm.at[pl.ds(base + i * CHUNK, CHUNK)],
                                  dis[b4], sds[b4]).wait()

        def _sca_start(b2, b4):
            pltpu.async_copy(ones_v, acc.at[dis[b4]], scs[b2], add=True)

        def _sca_wait(b2, b4):
            pltpu.make_async_copy(ones_v, acc.at[dis[b4]], scs[b2]).wait()

        _idx_start(0, 0)
        _idx_start(1, 1)

        def _step(i, b2, b4, first, start_idx):
            _idx_wait(i, b4)
            if not first:
                _sca_wait(b2, (b4 + 2) % 4)
            _sca_start(b2, b4)
            if start_idx:
                _idx_start(i + 2, (b4 + 2) % 4)

        _step(0, 0, 0, True, True)
        _step(1, 1, 1, True, True)
        _step(2, 0, 2, False, True)
        _step(3, 1, 3, False, True)

        def _body(i4, carry):
            i = 4 + i4 * 4
            _step(i + 0, 0, 0, False, True)
            _step(i + 1, 1, 1, False, True)
            _step(i + 2, 0, 2, False, True)
            _step(i + 3, 1, 3, False, True)
            return carry

        lax.fori_loop(0, (MAIN - 4) // 4, _body, 0)
        for i in range(MAIN, NFULL):
            _step(i, i % 2, i % 4, False, False)
        _sca_wait(0, (NFULL - 2) % 4)
        _sca_wait(1, (NFULL - 1) % 4)

        pltpu.sync_copy(dst_hbm.at[pl.ds(base + NFULL * CHUNK, TAIL)], di_t)
        pltpu.sync_copy(ones_t, acc.at[di_t], add=True)

        plsc.subcore_barrier()
        pltpu.sync_copy(acc.at[pl.ds(s * RPS, RPS), :],
                        out_hbm.at[c, pl.ds(s * RPS, RPS), :])

    return k(dst)


def _seg_sc(src, dst, g):
    @functools.partial(
        pl.kernel,
        out_type=jax.ShapeDtypeStruct((NC, NPAD, D), jnp.float32),
        mesh=_sc_mesh(),
        scratch_types=[
            pltpu.VMEM((CHUNK,), jnp.int32),
            pltpu.VMEM((CHUNK,), jnp.int32),
            pltpu.VMEM((CHUNK,), jnp.int32),
            pltpu.VMEM((CHUNK,), jnp.int32),
            pltpu.VMEM((CHUNK,), jnp.int32),
            pltpu.VMEM((CHUNK,), jnp.int32),
            pltpu.VMEM((CHUNK,), jnp.int32),
            pltpu.VMEM((CHUNK,), jnp.int32),
            pltpu.VMEM((CHUNK, D), jnp.float32),
            pltpu.VMEM((CHUNK, D), jnp.float32),
            pltpu.VMEM((TAIL,), jnp.int32),
            pltpu.VMEM((TAIL,), jnp.int32),
            pltpu.VMEM((TAIL, D), jnp.float32),
            pltpu.VMEM_SHARED((NPAD, D), jnp.float32),
            pltpu.SemaphoreType.DMA,
            pltpu.SemaphoreType.DMA,
            pltpu.SemaphoreType.DMA,
            pltpu.SemaphoreType.DMA,
            pltpu.SemaphoreType.DMA,
            pltpu.SemaphoreType.DMA,
            pltpu.SemaphoreType.DMA,
            pltpu.SemaphoreType.DMA,
            pltpu.SemaphoreType.DMA,
            pltpu.SemaphoreType.DMA,
            pltpu.SemaphoreType.DMA,
            pltpu.SemaphoreType.DMA,
        ],
    )
    def k(src_hbm, dst_hbm, g_hbm, out_hbm,
          si0, si1, si2, si3, di0, di1, di2, di3, rows0, rows1,
          si_t, di_t, rows_t, acc,
          sg0, sg1, sc0, sc1, ss0, ss1, ss2, ss3, sd0, sd1, sd2, sd3):
        c = lax.axis_index("c")
        s = lax.axis_index("s")
        base = (c * NS + s) * EPW
        sis, dis = (si0, si1, si2, si3), (di0, di1, di2, di3)
        rowss, sgs, scs = (rows0, rows1), (sg0, sg1), (sc0, sc1)
        sss, sds = (ss0, ss1, ss2, ss3), (sd0, sd1, sd2, sd3)

        def _fill(r, carry):
            for q in range(D // 16):
                rows0[r, pl.ds(q * 16, 16)] = jnp.zeros((16,), jnp.float32)
            return carry

        lax.fori_loop(0, CHUNK, _fill, 0)
        for j in range(RPS // CHUNK):
            pltpu.sync_copy(rows0, acc.at[pl.ds(s * RPS + j * CHUNK, CHUNK), :])
        plsc.subcore_barrier()

        def _idx_start(i, b4):
            off = base + i * CHUNK
            pltpu.async_copy(src_hbm.at[pl.ds(off, CHUNK)], sis[b4], sss[b4])
            pltpu.async_copy(dst_hbm.at[pl.ds(off, CHUNK)], dis[b4], sds[b4])

        def _idx_wait(i, b4):
            off = base + i * CHUNK
            pltpu.make_async_copy(src_hbm.at[pl.ds(off, CHUNK)], sis[b4],
                                  sss[b4]).wait()
            pltpu.make_async_copy(dst_hbm.at[pl.ds(off, CHUNK)], dis[b4],
                                  sds[b4]).wait()

        def _sca_start(b2, b4):
            pltpu.async_copy(rowss[b2], acc.at[dis[b4]], scs[b2], add=True)

        def _sca_wait(b2, b4):
            pltpu.make_async_copy(rowss[b2], acc.at[dis[b4]], scs[b2]).wait()

        _idx_start(0, 0)
        _idx_start(1, 1)
        _idx_start(2, 2)
        _idx_wait(0, 0)
        pltpu.async_copy(g_hbm.at[sis[0]], rowss[0], sgs[0])

        def _step(i, b2, b4, first, has_next, start_idx):
            nb2 = 1 - b2
            pltpu.make_async_copy(g_hbm.at[sis[b4]], rowss[b2], sgs[b2]).wait()
            if not first:
                _sca_wait(nb2, (b4 + 3) % 4)

            if has_next:
                _idx_wait(i + 1, (b4 + 1) % 4)
                pltpu.async_copy(g_hbm.at[sis[(b4 + 1) % 4]], rowss[nb2],
                                 sgs[nb2])

            _sca_start(b2, b4)

            if start_idx == "when":
                @pl.when(i + 3 < NFULL)
                def _():
                    _idx_start(i + 3, (b4 + 3) % 4)
            elif start_idx:
                _idx_start(i + 3, (b4 + 3) % 4)

        _step(0, 0, 0, True, True, True)
        _step(1, 1, 1, False, True, True)
        _step(2, 0, 2, False, True, True)
        _step(3, 1, 3, False, True, True)

        def _body(i4, carry):
            i = 4 + i4 * 4
            _step(i + 0, 0, 0, False, True, "when")
            _step(i + 1, 1, 1, False, True, "when")
            _step(i + 2, 0, 2, False, True, "when")
            _step(i + 3, 1, 3, False, True, "when")
            return carry

        lax.fori_loop(0, (MAIN - 4) // 4, _body, 0)
        for i in range(MAIN, NFULL):
            _step(i, i % 2, i % 4, False, i + 1 < NFULL, False)
        _sca_wait((NFULL - 1) % 2, (NFULL - 1) % 4)

        offt = base + NFULL * CHUNK
        pltpu.sync_copy(src_hbm.at[pl.ds(offt, TAIL)], si_t)
        pltpu.sync_copy(dst_hbm.at[pl.ds(offt, TAIL)], di_t)
        pltpu.async_copy(g_hbm.at[si_t], rows_t, sgs[0]).wait()
        pltpu.sync_copy(rows_t, acc.at[di_t], add=True)

        plsc.subcore_barrier()
        pltpu.sync_copy(acc.at[pl.ds(s * RPS, RPS), :],
                        out_hbm.at[c, pl.ds(s * RPS, RPS), :])

    return k(src, dst, g)


def _tc1_body(degp_ref, x_ref, w1_ref, dinv_ref, g1_ref):
    deg = degp_ref[0, :N, 0:1] + degp_ref[1, :N, 0:1] + 1.0
    dinv = lax.rsqrt(deg)
    dinv_ref[...] = dinv
    h = jnp.dot(x_ref[...], w1_ref[...], preferred_element_type=jnp.float32)
    g1_ref[...] = dinv * h


def _tc_first_body(g_ref, segp_ref, dinv_ref, b_ref, w_ref, h_ref, gn_ref):
    seg = segp_ref[0, :N, :] + segp_ref[1, :N, :]
    dinv = dinv_ref[...]
    conv = dinv * (g_ref[...] + seg) + b_ref[...]
    h = jnp.maximum(conv, 0.0)
    h_ref[...] = h
    gn_ref[...] = dinv * jnp.dot(h, w_ref[...], preferred_element_type=jnp.float32)


def _tc_mid_body(hp_ref, g_ref, segp_ref, dinv_ref, b_ref, w_ref, h_ref, gn_ref):
    seg = segp_ref[0, :N, :] + segp_ref[1, :N, :]
    dinv = dinv_ref[...]
    conv = dinv * (g_ref[...] + seg) + b_ref[...]
    h = jnp.maximum(hp_ref[...] + conv, 0.0)
    h_ref[...] = h
    gn_ref[...] = dinv * jnp.dot(h, w_ref[...], preferred_element_type=jnp.float32)


def _tc_out_body(hp_ref, g_ref, segp_ref, dinv_ref, b_ref,
                 wh1_ref, bh1_ref, wh2_ref, bh2_ref, out_ref):
    seg = segp_ref[0, :N, :] + segp_ref[1, :N, :]
    dinv = dinv_ref[...]
    conv = dinv * (g_ref[...] + seg) + b_ref[...]
    h = jnp.maximum(hp_ref[...] + conv, 0.0)
    z = jnp.dot(h, wh1_ref[...], preferred_element_type=jnp.float32) + bh1_ref[...]
    out_ref[...] = jnp.dot(z, wh2_ref[...], preferred_element_type=jnp.float32) + bh2_ref[...]


def _sds(shape):
    return jax.ShapeDtypeStruct(shape, jnp.float32)


def kernel(x, edge_index, W1, b1, W2, b2, W3, b3, Wh1, bh1, Wh2, bh2):
    src = edge_index[0]
    dst = edge_index[1]
    degp = _deg_sc(dst)
    dinv, g1 = pl.pallas_call(
        _tc1_body, out_shape=[_sds((N, 1)), _sds((N, D))],
    )(degp, x, W1)
    segp1 = _seg_sc(src, dst, g1)
    h1, g2 = pl.pallas_call(
        _tc_first_body, out_shape=[_sds((N, D)), _sds((N, D))],
    )(g1, segp1, dinv, b1, W2)
    segp2 = _seg_sc(src, dst, g2)
    h2, g3 = pl.pallas_call(
        _tc_mid_body, out_shape=[_sds((N, D)), _sds((N, D))],
    )(h1, g2, segp2, dinv, b2, W3)
    segp3 = _seg_sc(src, dst, g3)
    out = pl.pallas_call(
        _tc_out_body, out_shape=_sds((N, 1)),
    )(h2, g3, segp3, dinv, b3, Wh1, bh1, Wh2, bh2)
    return out


# speedup vs baseline: 24.9790x; 1.0863x over previous
import functools

import jax
import jax.numpy as jnp
from jax import lax
from jax.experimental import pallas as pl
from jax.experimental.pallas import tpu as pltpu
from jax.experimental.pallas import tpu_sc as plsc

N = 10000
E = 320000
D = 128

NC = 2
NS = 16
NW = NC * NS
NPAD = 10240
RPS = NPAD // NS
EPW = E // NW
CHUNK = 128
NFULL = EPW // CHUNK
TAIL = EPW - NFULL * CHUNK
MAIN = (NFULL // 4) * 4
DEGW = 128


def _sc_mesh():
    return plsc.VectorSubcoreMesh(core_axis_name="c", subcore_axis_name="s")


def _deg_sc(dst):
    @functools.partial(
        pl.kernel,
        out_type=jax.ShapeDtypeStruct((NC, NPAD, DEGW), jnp.float32),
        mesh=_sc_mesh(),
        scratch_types=[
            pltpu.VMEM((CHUNK,), jnp.int32),
            pltpu.VMEM((CHUNK,), jnp.int32),
            pltpu.VMEM((CHUNK,), jnp.int32),
            pltpu.VMEM((CHUNK,), jnp.int32),
            pltpu.VMEM((TAIL,), jnp.int32),
            pltpu.VMEM((CHUNK, DEGW), jnp.float32),
            pltpu.VMEM((TAIL, DEGW), jnp.float32),
            pltpu.VMEM_SHARED((NPAD, DEGW), jnp.float32),
            pltpu.SemaphoreType.DMA,
            pltpu.SemaphoreType.DMA,
            pltpu.SemaphoreType.DMA,
            pltpu.SemaphoreType.DMA,
            pltpu.SemaphoreType.DMA,
            pltpu.SemaphoreType.DMA,
        ],
    )
    def k(dst_hbm, out_hbm, di0, di1, di2, di3, di_t, ones_v, ones_t, acc,
          sc0, sc1, sd0, sd1, sd2, sd3):
        c = lax.axis_index("c")
        s = lax.axis_index("s")
        base = (c * NS + s) * EPW
        dis, sds, scs = (di0, di1, di2, di3), (sd0, sd1, sd2, sd3), (sc0, sc1)

        e0 = jnp.where(lax.iota(jnp.int32, 16) == 0,
                       jnp.float32(1.0), jnp.float32(0.0))

        def _fill(r, carry):
            for q in range(DEGW // 16):
                ones_v[r, pl.ds(q * 16, 16)] = e0 if q == 0 else jnp.zeros(
                    (16,), jnp.float32)
            return carry

        lax.fori_loop(0, CHUNK, _fill, 0)
        for r in range(TAIL):
            for q in range(DEGW // 16):
                ones_t[r, pl.ds(q * 16, 16)] = e0 if q == 0 else jnp.zeros(
                    (16,), jnp.float32)

        def _zrow(r, carry):
            for q in range(DEGW // 16):
                ones_t[r, pl.ds(q * 16, 16)] = jnp.zeros((16,), jnp.float32)
            return carry

        lax.fori_loop(0, TAIL, _zrow, 0)
        for j in range(RPS // TAIL):
            pltpu.sync_copy(ones_t, acc.at[pl.ds(s * RPS + j * TAIL, TAIL), :])

        for r in range(TAIL):
            ones_t[r, pl.ds(0, 16)] = e0
        plsc.subcore_barrier()

        def _idx_start(i, b4):
            pltpu.async_copy(dst_hbm.at[pl.ds(base + i * CHUNK, CHUNK)],
                             dis[b4], sds[b4])

        def _idx_wait(i, b4):
            pltpu.make_async_copy(dst_hbm.at[pl.ds(base + i * CHUNK, CHUNK)],
                                  dis[b4], sds[b4]).wait()

        def _sca_start(b2, b4):
            pltpu.async_copy(ones_v, acc.at[dis[b4]], scs[b2], add=True)

        def _sca_wait(b2, b4):
            pltpu.make_async_copy(ones_v, acc.at[dis[b4]], scs[b2]).wait()

        _idx_start(0, 0)
        _idx_start(1, 1)

        def _step(i, b2, b4, first, start_idx):
            _idx_wait(i, b4)
            if not first:
                _sca_wait(b2, (b4 + 2) % 4)
            _sca_start(b2, b4)
            if start_idx:
                _idx_start(i + 2, (b4 + 2) % 4)

        _step(0, 0, 0, True, True)
        _step(1, 1, 1, True, True)
        _step(2, 0, 2, False, True)
        _step(3, 1, 3, False, True)

        def _body(i4, carry):
            i = 4 + i4 * 4
            _step(i + 0, 0, 0, False, True)
            _step(i + 1, 1, 1, False, True)
            _step(i + 2, 0, 2, False, True)
            _step(i + 3, 1, 3, False, True)
            return carry

        lax.fori_loop(0, (MAIN - 4) // 4, _body, 0)
        for i in range(MAIN, NFULL):
            _step(i, i % 2, i % 4, False, False)
        _sca_wait(0, (NFULL - 2) % 4)
        _sca_wait(1, (NFULL - 1) % 4)

        pltpu.sync_copy(dst_hbm.at[pl.ds(base + NFULL * CHUNK, TAIL)], di_t)
        pltpu.sync_copy(ones_t, acc.at[di_t], add=True)

        plsc.subcore_barrier()
        pltpu.sync_copy(acc.at[pl.ds(s * RPS, RPS), :],
                        out_hbm.at[c, pl.ds(s * RPS, RPS), :])

    return k(dst)


SCH = 80
SNF = EPW // SCH


def _seg_sc(src, dst, g):
    idx_scratch = [pltpu.VMEM((SCH,), jnp.int32) for _ in range(16)]
    row_scratch = [pltpu.VMEM((SCH, D), jnp.float32) for _ in range(4)]
    sem_scratch = [pltpu.SemaphoreType.DMA for _ in range(24)]

    @functools.partial(
        pl.kernel,
        out_type=jax.ShapeDtypeStruct((NC, NPAD, D), jnp.float32),
        mesh=_sc_mesh(),
        scratch_types=idx_scratch + row_scratch
        + [pltpu.VMEM_SHARED((NPAD, D), jnp.float32)] + sem_scratch,
    )
    def k(src_hbm, dst_hbm, g_hbm, out_hbm, *scr):
        sis = scr[0:8]
        dis = scr[8:16]
        rowss = scr[16:20]
        acc = scr[20]
        sgs = scr[21:25]
        scs = scr[25:29]
        sss = scr[29:37]
        sds = scr[37:45]
        c = lax.axis_index("c")
        s = lax.axis_index("s")
        base = (c * NS + s) * EPW

        def _fill(r, carry):
            for q in range(D // 16):
                rowss[0][r, pl.ds(q * 16, 16)] = jnp.zeros((16,), jnp.float32)
            return carry

        lax.fori_loop(0, SCH, _fill, 0)
        for j in range(RPS // SCH):
            pltpu.sync_copy(rowss[0], acc.at[pl.ds(s * RPS + j * SCH, SCH), :])
        plsc.subcore_barrier()

        def _idx_start(i, b8):
            off = base + i * SCH
            pltpu.async_copy(src_hbm.at[pl.ds(off, SCH)], sis[b8], sss[b8])
            pltpu.async_copy(dst_hbm.at[pl.ds(off, SCH)], dis[b8], sds[b8])

        def _idx_wait(i, b8):
            off = base + i * SCH
            pltpu.make_async_copy(src_hbm.at[pl.ds(off, SCH)], sis[b8],
                                  sss[b8]).wait()
            pltpu.make_async_copy(dst_hbm.at[pl.ds(off, SCH)], dis[b8],
                                  sds[b8]).wait()

        def _g_start(b8, b4):
            pltpu.async_copy(g_hbm.at[sis[b8]], rowss[b4], sgs[b4])

        def _g_wait(b8, b4):
            pltpu.make_async_copy(g_hbm.at[sis[b8]], rowss[b4], sgs[b4]).wait()

        def _sc_start(b8, b4):
            pltpu.async_copy(rowss[b4], acc.at[dis[b8]], scs[b4], add=True)

        def _sc_wait(b8, b4):
            pltpu.make_async_copy(rowss[b4], acc.at[dis[b8]], scs[b4]).wait()

        for i in range(6):
            _idx_start(i, i)
        _idx_wait(0, 0)
        _g_start(0, 0)
        _idx_wait(1, 1)
        _g_start(1, 1)

        def _step(i, b8, b4, first, do_g2, do_idx):
            _g_wait(b8, b4)
            if not first:
                _sc_wait((b8 + 6) % 8, (b4 + 2) % 4)
            if do_g2:
                _idx_wait(i + 2, (b8 + 2) % 8)
                _g_start((b8 + 2) % 8, (b4 + 2) % 4)
            _sc_start(b8, b4)
            if do_idx == "when":
                @pl.when(i + 6 < SNF)
                def _():
                    _idx_start(i + 6, (b8 + 6) % 8)
            elif do_idx:
                _idx_start(i + 6, (b8 + 6) % 8)

        for i in range(8):
            _step(i, i % 8, i % 4, i < 2, True, True)

        def _body(i8, carry):
            i = 8 + i8 * 8
            for u in range(8):
                _step(i + u, u % 8, u % 4, False, True, "when")
            return carry

        lax.fori_loop(0, (SNF - 8) // 8 - 1, _body, 0)
        for i in range(8 + ((SNF - 8) // 8 - 1) * 8, SNF):
            _step(i, i % 8, i % 4, False, i + 2 < SNF, i + 6 < SNF)
        _sc_wait((SNF - 2) % 8, (SNF - 2) % 4)
        _sc_wait((SNF - 1) % 8, (SNF - 1) % 4)

        plsc.subcore_barrier()
        pltpu.sync_copy(acc.at[pl.ds(s * RPS, RPS), :],
                        out_hbm.at[c, pl.ds(s * RPS, RPS), :])

    return k(src, dst, g)


def _tc1_body(degp_ref, x_ref, w1_ref, dinv_ref, g1_ref):
    deg = degp_ref[0, :N, 0:1] + degp_ref[1, :N, 0:1] + 1.0
    dinv = lax.rsqrt(deg)
    dinv_ref[...] = dinv
    h = jnp.dot(x_ref[...], w1_ref[...], preferred_element_type=jnp.float32)
    g1_ref[...] = dinv * h


def _tc_first_body(g_ref, segp_ref, dinv_ref, b_ref, w_ref, h_ref, gn_ref):
    seg = segp_ref[0, :N, :] + segp_ref[1, :N, :]
    dinv = dinv_ref[...]
    conv = dinv * (g_ref[...] + seg) + b_ref[...]
    h = jnp.maximum(conv, 0.0)
    h_ref[...] = h
    gn_ref[...] = dinv * jnp.dot(h, w_ref[...], preferred_element_type=jnp.float32)


def _tc_mid_body(hp_ref, g_ref, segp_ref, dinv_ref, b_ref, w_ref, h_ref, gn_ref):
    seg = segp_ref[0, :N, :] + segp_ref[1, :N, :]
    dinv = dinv_ref[...]
    conv = dinv * (g_ref[...] + seg) + b_ref[...]
    h = jnp.maximum(hp_ref[...] + conv, 0.0)
    h_ref[...] = h
    gn_ref[...] = dinv * jnp.dot(h, w_ref[...], preferred_element_type=jnp.float32)


def _tc_out_body(hp_ref, g_ref, segp_ref, dinv_ref, b_ref,
                 wh1_ref, bh1_ref, wh2_ref, bh2_ref, out_ref):
    seg = segp_ref[0, :N, :] + segp_ref[1, :N, :]
    dinv = dinv_ref[...]
    conv = dinv * (g_ref[...] + seg) + b_ref[...]
    h = jnp.maximum(hp_ref[...] + conv, 0.0)
    z = jnp.dot(h, wh1_ref[...], preferred_element_type=jnp.float32) + bh1_ref[...]
    out_ref[...] = jnp.dot(z, wh2_ref[...], preferred_element_type=jnp.float32) + bh2_ref[...]


def _sds(shape):
    return jax.ShapeDtypeStruct(shape, jnp.float32)


def kernel(x, edge_index, W1, b1, W2, b2, W3, b3, Wh1, bh1, Wh2, bh2):
    src = edge_index[0]
    dst = edge_index[1]
    degp = _deg_sc(dst)
    dinv, g1 = pl.pallas_call(
        _tc1_body, out_shape=[_sds((N, 1)), _sds((N, D))],
    )(degp, x, W1)
    segp1 = _seg_sc(src, dst, g1)
    h1, g2 = pl.pallas_call(
        _tc_first_body, out_shape=[_sds((N, D)), _sds((N, D))],
    )(g1, segp1, dinv, b1, W2)
    segp2 = _seg_sc(src, dst, g2)
    h2, g3 = pl.pallas_call(
        _tc_mid_body, out_shape=[_sds((N, D)), _sds((N, D))],
    )(h1, g2, segp2, dinv, b2, W3)
    segp3 = _seg_sc(src, dst, g3)
    out = pl.pallas_call(
        _tc_out_body, out_shape=_sds((N, 1)),
    )(h2, g3, segp3, dinv, b3, Wh1, bh1, Wh2, bh2)
    return out


# speedup vs baseline: 25.1094x; 1.0052x over previous
import functools

import jax
import jax.numpy as jnp
from jax import lax
from jax.experimental import pallas as pl
from jax.experimental.pallas import tpu as pltpu
from jax.experimental.pallas import tpu_sc as plsc

N = 10000
E = 320000
D = 128

NC = 2
NS = 16
NW = NC * NS
NPAD = 10240
RPS = NPAD // NS
EPW = E // NW
CHUNK = 128
NFULL = EPW // CHUNK
TAIL = EPW - NFULL * CHUNK
MAIN = (NFULL // 4) * 4
DEGW = 128


def _sc_mesh():
    return plsc.VectorSubcoreMesh(core_axis_name="c", subcore_axis_name="s")


def _deg_sc(dst):
    @functools.partial(
        pl.kernel,
        out_type=jax.ShapeDtypeStruct((NC, NPAD, DEGW), jnp.float32),
        mesh=_sc_mesh(),
        scratch_types=[
            pltpu.VMEM((CHUNK,), jnp.int32),
            pltpu.VMEM((CHUNK,), jnp.int32),
            pltpu.VMEM((CHUNK,), jnp.int32),
            pltpu.VMEM((CHUNK,), jnp.int32),
            pltpu.VMEM((TAIL,), jnp.int32),
            pltpu.VMEM((CHUNK, DEGW), jnp.float32),
            pltpu.VMEM((TAIL, DEGW), jnp.float32),
            pltpu.VMEM_SHARED((NPAD, DEGW), jnp.float32),
            pltpu.SemaphoreType.DMA,
            pltpu.SemaphoreType.DMA,
            pltpu.SemaphoreType.DMA,
            pltpu.SemaphoreType.DMA,
            pltpu.SemaphoreType.DMA,
            pltpu.SemaphoreType.DMA,
        ],
    )
    def k(dst_hbm, out_hbm, di0, di1, di2, di3, di_t, ones_v, ones_t, acc,
          sc0, sc1, sd0, sd1, sd2, sd3):
        c = lax.axis_index("c")
        s = lax.axis_index("s")
        base = (c * NS + s) * EPW
        dis, sds, scs = (di0, di1, di2, di3), (sd0, sd1, sd2, sd3), (sc0, sc1)

        e0 = jnp.where(lax.iota(jnp.int32, 16) == 0,
                       jnp.float32(1.0), jnp.float32(0.0))

        def _idx_start0(i, b4):
            pltpu.async_copy(dst_hbm.at[pl.ds(base + i * CHUNK, CHUNK)],
                             dis[b4], sds[b4])

        _idx_start0(0, 0)
        _idx_start0(1, 1)

        def _fill(r, carry):
            for q in range(DEGW // 16):
                ones_v[r, pl.ds(q * 16, 16)] = e0 if q == 0 else jnp.zeros(
                    (16,), jnp.float32)
            return carry

        lax.fori_loop(0, CHUNK, _fill, 0)
        for r in range(TAIL):
            for q in range(DEGW // 16):
                ones_t[r, pl.ds(q * 16, 16)] = e0 if q == 0 else jnp.zeros(
                    (16,), jnp.float32)

        def _zrow(r, carry):
            for q in range(DEGW // 16):
                ones_t[r, pl.ds(q * 16, 16)] = jnp.zeros((16,), jnp.float32)
            return carry

        lax.fori_loop(0, TAIL, _zrow, 0)
        for j in range(RPS // TAIL):
            pltpu.sync_copy(ones_t, acc.at[pl.ds(s * RPS + j * TAIL, TAIL), :])

        for r in range(TAIL):
            ones_t[r, pl.ds(0, 16)] = e0
        plsc.subcore_barrier()

        _idx_start = _idx_start0

        def _idx_wait(i, b4):
            pltpu.make_async_copy(dst_hbm.at[pl.ds(base + i * CHUNK, CHUNK)],
                                  dis[b4], sds[b4]).wait()

        def _sca_start(b2, b4):
            pltpu.async_copy(ones_v, acc.at[dis[b4]], scs[b2], add=True)

        def _sca_wait(b2, b4):
            pltpu.make_async_copy(ones_v, acc.at[dis[b4]], scs[b2]).wait()

        def _step(i, b2, b4, first, start_idx):
            _idx_wait(i, b4)
            if not first:
                _sca_wait(b2, (b4 + 2) % 4)
            _sca_start(b2, b4)
            if start_idx:
                _idx_start(i + 2, (b4 + 2) % 4)

        _step(0, 0, 0, True, True)
        _step(1, 1, 1, True, True)
        _step(2, 0, 2, False, True)
        _step(3, 1, 3, False, True)

        def _body(i4, carry):
            i = 4 + i4 * 4
            _step(i + 0, 0, 0, False, True)
            _step(i + 1, 1, 1, False, True)
            _step(i + 2, 0, 2, False, True)
            _step(i + 3, 1, 3, False, True)
            return carry

        lax.fori_loop(0, (MAIN - 4) // 4, _body, 0)
        for i in range(MAIN, NFULL):
            _step(i, i % 2, i % 4, False, False)
        _sca_wait(0, (NFULL - 2) % 4)
        _sca_wait(1, (NFULL - 1) % 4)

        pltpu.sync_copy(dst_hbm.at[pl.ds(base + NFULL * CHUNK, TAIL)], di_t)
        pltpu.sync_copy(ones_t, acc.at[di_t], add=True)

        plsc.subcore_barrier()
        pltpu.sync_copy(acc.at[pl.ds(s * RPS, RPS), :],
                        out_hbm.at[c, pl.ds(s * RPS, RPS), :])

    return k(dst)


SCH = 80
SNF = EPW // SCH


def _seg_sc(src, dst, g):
    idx_scratch = [pltpu.VMEM((SCH,), jnp.int32) for _ in range(16)]
    row_scratch = [pltpu.VMEM((SCH, D), jnp.float32) for _ in range(4)]
    sem_scratch = [pltpu.SemaphoreType.DMA for _ in range(24)]

    @functools.partial(
        pl.kernel,
        out_type=jax.ShapeDtypeStruct((NC, NPAD, D), jnp.float32),
        mesh=_sc_mesh(),
        scratch_types=idx_scratch + row_scratch
        + [pltpu.VMEM_SHARED((NPAD, D), jnp.float32)] + sem_scratch,
    )
    def k(src_hbm, dst_hbm, g_hbm, out_hbm, *scr):
        sis = scr[0:8]
        dis = scr[8:16]
        rowss = scr[16:20]
        acc = scr[20]
        sgs = scr[21:25]
        scs = scr[25:29]
        sss = scr[29:37]
        sds = scr[37:45]
        c = lax.axis_index("c")
        s = lax.axis_index("s")
        base = (c * NS + s) * EPW

        def _idx_start(i, b8):
            off = base + i * SCH
            pltpu.async_copy(src_hbm.at[pl.ds(off, SCH)], sis[b8], sss[b8])
            pltpu.async_copy(dst_hbm.at[pl.ds(off, SCH)], dis[b8], sds[b8])

        def _idx_wait(i, b8):
            off = base + i * SCH
            pltpu.make_async_copy(src_hbm.at[pl.ds(off, SCH)], sis[b8],
                                  sss[b8]).wait()
            pltpu.make_async_copy(dst_hbm.at[pl.ds(off, SCH)], dis[b8],
                                  sds[b8]).wait()

        def _g_start(b8, b4):
            pltpu.async_copy(g_hbm.at[sis[b8]], rowss[b4], sgs[b4])

        def _g_wait(b8, b4):
            pltpu.make_async_copy(g_hbm.at[sis[b8]], rowss[b4], sgs[b4]).wait()

        def _sc_start(b8, b4):
            pltpu.async_copy(rowss[b4], acc.at[dis[b8]], scs[b4], add=True)

        def _sc_wait(b8, b4):
            pltpu.make_async_copy(rowss[b4], acc.at[dis[b8]], scs[b4]).wait()

        for i in range(6):
            _idx_start(i, i)

        def _fill(r, carry):
            for q in range(D // 16):
                rowss[0][r, pl.ds(q * 16, 16)] = jnp.zeros((16,), jnp.float32)
            return carry

        lax.fori_loop(0, SCH, _fill, 0)
        for j in range(RPS // SCH):
            pltpu.sync_copy(rowss[0], acc.at[pl.ds(s * RPS + j * SCH, SCH), :])
        plsc.subcore_barrier()

        _idx_wait(0, 0)
        _g_start(0, 0)
        _idx_wait(1, 1)
        _g_start(1, 1)

        def _step(i, b8, b4, first, do_g2, do_idx):
            _g_wait(b8, b4)
            if not first:
                _sc_wait((b8 + 6) % 8, (b4 + 2) % 4)
            if do_g2:
                _idx_wait(i + 2, (b8 + 2) % 8)
                _g_start((b8 + 2) % 8, (b4 + 2) % 4)
            _sc_start(b8, b4)
            if do_idx == "when":
                @pl.when(i + 6 < SNF)
                def _():
                    _idx_start(i + 6, (b8 + 6) % 8)
            elif do_idx:
                _idx_start(i + 6, (b8 + 6) % 8)

        for i in range(8):
            _step(i, i % 8, i % 4, i < 2, True, True)

        def _body(i8, carry):
            i = 8 + i8 * 8
            for u in range(8):
                _step(i + u, u % 8, u % 4, False, True, "when")
            return carry

        lax.fori_loop(0, (SNF - 8) // 8 - 1, _body, 0)
        for i in range(8 + ((SNF - 8) // 8 - 1) * 8, SNF):
            _step(i, i % 8, i % 4, False, i + 2 < SNF, i + 6 < SNF)
        _sc_wait((SNF - 2) % 8, (SNF - 2) % 4)
        _sc_wait((SNF - 1) % 8, (SNF - 1) % 4)

        plsc.subcore_barrier()
        pltpu.sync_copy(acc.at[pl.ds(s * RPS, RPS), :],
                        out_hbm.at[c, pl.ds(s * RPS, RPS), :])

    return k(src, dst, g)


def _tc1_body(degp_ref, x_ref, w1_ref, dinv_ref, g1_ref):
    deg = degp_ref[0, :N, 0:1] + degp_ref[1, :N, 0:1] + 1.0
    dinv = lax.rsqrt(deg)
    dinv_ref[...] = dinv
    h = jnp.dot(x_ref[...], w1_ref[...], preferred_element_type=jnp.float32)
    g1_ref[...] = dinv * h


def _tc_first_body(g_ref, segp_ref, dinv_ref, b_ref, w_ref, h_ref, gn_ref):
    seg = segp_ref[0, :N, :] + segp_ref[1, :N, :]
    dinv = dinv_ref[...]
    conv = dinv * (g_ref[...] + seg) + b_ref[...]
    h = jnp.maximum(conv, 0.0)
    h_ref[...] = h
    gn_ref[...] = dinv * jnp.dot(h, w_ref[...], preferred_element_type=jnp.float32)


def _tc_mid_body(hp_ref, g_ref, segp_ref, dinv_ref, b_ref, w_ref, h_ref, gn_ref):
    seg = segp_ref[0, :N, :] + segp_ref[1, :N, :]
    dinv = dinv_ref[...]
    conv = dinv * (g_ref[...] + seg) + b_ref[...]
    h = jnp.maximum(hp_ref[...] + conv, 0.0)
    h_ref[...] = h
    gn_ref[...] = dinv * jnp.dot(h, w_ref[...], preferred_element_type=jnp.float32)


def _tc_out_body(hp_ref, g_ref, segp_ref, dinv_ref, b_ref,
                 wh1_ref, bh1_ref, wh2_ref, bh2_ref, out_ref):
    seg = segp_ref[0, :N, :] + segp_ref[1, :N, :]
    dinv = dinv_ref[...]
    conv = dinv * (g_ref[...] + seg) + b_ref[...]
    h = jnp.maximum(hp_ref[...] + conv, 0.0)
    z = jnp.dot(h, wh1_ref[...], preferred_element_type=jnp.float32) + bh1_ref[...]
    out_ref[...] = jnp.dot(z, wh2_ref[...], preferred_element_type=jnp.float32) + bh2_ref[...]


def _sds(shape):
    return jax.ShapeDtypeStruct(shape, jnp.float32)


def kernel(x, edge_index, W1, b1, W2, b2, W3, b3, Wh1, bh1, Wh2, bh2):
    src = edge_index[0]
    dst = edge_index[1]
    degp = _deg_sc(dst)
    dinv, g1 = pl.pallas_call(
        _tc1_body, out_shape=[_sds((N, 1)), _sds((N, D))],
    )(degp, x, W1)
    segp1 = _seg_sc(src, dst, g1)
    h1, g2 = pl.pallas_call(
        _tc_first_body, out_shape=[_sds((N, D)), _sds((N, D))],
    )(g1, segp1, dinv, b1, W2)
    segp2 = _seg_sc(src, dst, g2)
    h2, g3 = pl.pallas_call(
        _tc_mid_body, out_shape=[_sds((N, D)), _sds((N, D))],
    )(h1, g2, segp2, dinv, b2, W3)
    segp3 = _seg_sc(src, dst, g3)
    out = pl.pallas_call(
        _tc_out_body, out_shape=_sds((N, 1)),
    )(h2, g3, segp3, dinv, b3, Wh1, bh1, Wh2, bh2)
    return out
